# baseline (speedup 1.0000x reference)
import jax
import jax.numpy as jnp
from jax.experimental import pallas as pl

_BK = 12544
_THRESH_SQ = 0.25
_NTOP = 100.0


def _count_kernel(q_ref, r_ref, o_ref):
    k = pl.program_id(0)
    nk = pl.num_programs(0)
    q = q_ref[...]
    rt = jnp.transpose(r_ref[...])
    ones_q = jnp.ones((q.shape[0], 1), jnp.float32)
    q2 = jnp.sum(q * q, axis=1, keepdims=True)
    qa = jnp.concatenate([-2.0 * q, q2, ones_q], axis=1)
    ones_r = jnp.ones((1, rt.shape[1]), jnp.float32)
    r2 = jnp.sum(rt * rt, axis=0, keepdims=True)
    ra = jnp.concatenate([rt, ones_r, r2], axis=0)
    d2 = jnp.dot(qa, ra, preferred_element_type=jnp.float32)
    cnt = jnp.sum((d2 <= _THRESH_SQ).astype(jnp.float32), axis=1, keepdims=True)

    @pl.when(k == 0)
    def _init():
        o_ref[...] = jnp.zeros_like(o_ref)

    o_ref[...] += cnt

    @pl.when(k == nk - 1)
    def _finish():
        o_ref[...] = jnp.minimum(o_ref[...], _NTOP) * (1.0 / _NTOP)


@jax.jit
def _run(q, r):
    Q, D = q.shape
    K = r.shape[0]
    Kp = ((K + _BK - 1) // _BK) * _BK
    if Kp != K:
        r = jnp.pad(r, ((0, Kp - K), (0, 0)), constant_values=1e6)
    nk = Kp // _BK
    out = pl.pallas_call(
        _count_kernel,
        grid=(nk,),
        in_specs=[
            pl.BlockSpec((Q, D), lambda k: (0, 0)),
            pl.BlockSpec((_BK, D), lambda k: (k, 0)),
        ],
        out_specs=pl.BlockSpec((Q, 1), lambda k: (0, 0)),
        out_shape=jax.ShapeDtypeStruct((Q, 1), jnp.float32),
    )(q, r)
    return out[:, 0]


def kernel(query_embeddings, reference_embeddings):
    return _run(query_embeddings, reference_embeddings)


# speedup vs baseline: 1.8269x; 1.8269x over previous
import jax
import jax.numpy as jnp
from jax.experimental import pallas as pl

_BK = 14336
_THRESH_SQ = 0.25
_NTOP = 100.0


def _count_kernel(q_ref, rt_ref, o_ref):
    k = pl.program_id(0)
    nk = pl.num_programs(0)
    q = q_ref[...]
    rt = rt_ref[...]
    ones_q = jnp.ones((q.shape[0], 1), jnp.float32)
    q2 = jnp.sum(q * q, axis=1, keepdims=True)
    qa = jnp.concatenate([-2.0 * q, q2, ones_q], axis=1)
    ones_r = jnp.ones((1, rt.shape[1]), jnp.float32)
    r2 = jnp.sum(rt * rt, axis=0, keepdims=True)
    ra = jnp.concatenate([rt, ones_r, r2], axis=0)
    d2 = jnp.dot(qa, ra, preferred_element_type=jnp.float32)
    cnt = jnp.sum((d2 <= _THRESH_SQ).astype(jnp.float32), axis=1, keepdims=True)

    @pl.when(k == 0)
    def _init():
        o_ref[...] = jnp.zeros_like(o_ref)

    o_ref[...] += cnt

    @pl.when(k == nk - 1)
    def _finish():
        o_ref[...] = jnp.minimum(o_ref[...], _NTOP) * (1.0 / _NTOP)


@jax.jit
def _run(q, r):
    Q, D = q.shape
    K = r.shape[0]
    Kp = ((K + _BK - 1) // _BK) * _BK
    rt = jnp.transpose(r)
    if Kp != K:
        rt = jnp.pad(rt, ((0, 0), (0, Kp - K)), constant_values=1e6)
    nk = Kp // _BK
    out = pl.pallas_call(
        _count_kernel,
        grid=(nk,),
        in_specs=[
            pl.BlockSpec((Q, D), lambda k: (0, 0)),
            pl.BlockSpec((D, _BK), lambda k: (0, k)),
        ],
        out_specs=pl.BlockSpec((Q, 1), lambda k: (0, 0)),
        out_shape=jax.ShapeDtypeStruct((Q, 1), jnp.float32),
    )(q, rt)
    return out[:, 0]


def kernel(query_embeddings, reference_embeddings):
    return _run(query_embeddings, reference_embeddings)
